# baseline (speedup 1.0000x reference)
import functools

import jax
import jax.numpy as jnp
from jax.experimental import pallas as pl


def _prep(qk):
    qk = qk.astype(jnp.float16)
    b, h, s, d = qk.shape
    qk_norm = jnp.sqrt(jnp.sum(qk * qk, axis=-1, keepdims=True))
    m = jnp.max(qk_norm)
    qk = qk * jnp.float16(0.75) / m
    tmp_zero = jnp.zeros(qk_norm.shape, dtype=jnp.float16)
    p = qk
    q = qk
    for i in range(2):
        tmp = jnp.float16(0.5) - jnp.power(qk_norm, (i + 1) * 2)
        p = jnp.concatenate((p, tmp), axis=-1)
        q = jnp.concatenate((q, tmp_zero), axis=-1)
    p_norm = jnp.sqrt(jnp.sum(p * p, axis=-1, keepdims=True))
    q_norm = jnp.sqrt(jnp.sum(q * q, axis=-1, keepdims=True))
    mm = jnp.max(p_norm)
    p = p / p_norm * mm
    q = q / q_norm * mm
    a = jax.random.normal(jax.random.key(42), (b, h, s, d + 2),
                          dtype=jnp.float32).astype(jnp.float16)
    qh = jnp.sum(q * a, axis=-1)
    return p, qh, a


_RB = 256
_K = 32


def _body(a_ref, pt_ref, qh_ref, o_ref):
    a_blk = a_ref[...]
    pt = pt_ref[...]
    qh = qh_ref[0:1, :]

    acc = jax.lax.dot_general(
        a_blk, pt, (((1,), (0,)), ((), ())),
        preferred_element_type=jnp.float32)
    s = acc * qh
    s = jnp.where(jnp.isnan(s), jnp.float32(0.0), s)
    s = s + jnp.float32(0.0)

    bits = jax.lax.bitcast_convert_type(s, jnp.int32)
    mono = jnp.where(bits < 0, ~bits, bits | jnp.int32(-2147483648))
    mono = jax.lax.bitcast_convert_type(mono, jnp.uint32)
    mono_s = jax.lax.bitcast_convert_type(mono, jnp.int32) ^ jnp.int32(-2147483648)

    rb, n = s.shape

    def count_ge(t):
        return jnp.sum((mono >= t).astype(jnp.int32), axis=1, keepdims=True)

    m = mono_s[:, 0:128]
    for k in range(1, 16):
        m = jnp.maximum(m, mono_s[:, k * 128:(k + 1) * 128])
    mm = jnp.maximum(jnp.maximum(m[:, 0:32], m[:, 32:64]),
                     jnp.maximum(m[:, 64:96], m[:, 96:128]))
    flip = jnp.int32(-2147483648)
    lo0 = jax.lax.bitcast_convert_type(
        jnp.min(mm, axis=1, keepdims=True) ^ flip, jnp.uint32)
    hi0 = jax.lax.bitcast_convert_type(
        jnp.max(m, axis=1, keepdims=True) ^ flip, jnp.uint32)

    clo0 = count_ge(lo0)
    chi0 = count_ge(hi0)

    def vcond(carry):
        lo, hi = carry[0], carry[1]
        return jnp.any(lo != hi)

    def vstep(carry):
        lo, hi, clo, chi, it = carry
        d = hi - lo
        bis = (d + jnp.uint32(1)) >> 1
        df = jax.lax.bitcast_convert_type(d, jnp.int32).astype(jnp.float32)
        denom = jnp.maximum((clo - chi).astype(jnp.float32), jnp.float32(1.0))
        off_f = df * ((clo - _K).astype(jnp.float32) / denom)
        off = jnp.clip(off_f.astype(jnp.int32), 1,
                       jax.lax.bitcast_convert_type(d, jnp.int32))
        use_interp = jnp.logical_and(it >= 2, (it & 1) == 0)
        step = jnp.where(use_interp,
                         jax.lax.bitcast_convert_type(off, jnp.uint32), bis)
        step = jnp.where(d == jnp.uint32(0), jnp.uint32(0), step)
        mid = lo + step
        cnt = count_ge(mid)
        ge = cnt >= _K
        return (jnp.where(ge, mid, lo), jnp.where(ge, hi, mid - jnp.uint32(1)),
                jnp.where(ge, cnt, clo), jnp.where(ge, chi, cnt), it + 1)

    v32, _, _, _, _ = jax.lax.while_loop(
        vcond, vstep, (lo0, hi0, clo0, chi0, jnp.int32(0)))

    col = jax.lax.broadcasted_iota(jnp.int32, s.shape, 1)
    tie_rows = jnp.any(count_ge(v32) > _K)

    def tie_search():
        need = _K - jnp.sum((mono > v32).astype(jnp.int32),
                            axis=1, keepdims=True)
        tie = mono == v32

        def cstep(_, carry):
            lo, hi = carry
            mid = (lo + hi) >> 1
            cnt = jnp.sum((tie & (col <= mid)).astype(jnp.int32),
                          axis=1, keepdims=True)
            ge = cnt >= need
            return jnp.where(ge, lo, mid + 1), jnp.where(ge, mid, hi)

        _, c32 = jax.lax.fori_loop(0, 11, cstep,
                                   (jnp.zeros((rb, 1), jnp.int32),
                                    jnp.full((rb, 1), n - 1, jnp.int32)))
        return c32

    c32 = jax.lax.cond(tie_rows, tie_search,
                       lambda: jnp.full((rb, 1), n - 1, jnp.int32))
    sel = (mono > v32) | ((mono == v32) & (col <= c32))
    o_ref[...] = jnp.where(sel, jnp.float32(0.0), jnp.float32(-10000.0))


def kernel(qk, bucket_size):
    del bucket_size
    b, h, s, d = qk.shape
    p, qh, a = _prep(qk)
    pad = 128 - (d + 2)
    pt = jnp.transpose(
        jnp.pad(p[0], ((0, 0), (0, 0), (0, pad))), (0, 2, 1))
    pt = pt.reshape(h * 128, s).astype(jnp.float32)
    a_pad = jnp.pad(a[0], ((0, 0), (0, 0), (0, pad)))
    a_pad = a_pad.reshape(h * s, 128).astype(jnp.float32)
    qh2 = jnp.broadcast_to(qh[0][:, None, :],
                           (h, 8, s)).reshape(h * 8, s).astype(jnp.float32)

    nr = s // _RB
    out = pl.pallas_call(
        _body,
        grid=(h, nr),
        in_specs=[
            pl.BlockSpec((_RB, 128), lambda i, r: (i * nr + r, 0)),
            pl.BlockSpec((128, s), lambda i, r: (i, 0)),
            pl.BlockSpec((8, s), lambda i, r: (i, 0)),
        ],
        out_specs=pl.BlockSpec((_RB, s), lambda i, r: (i * nr + r, 0)),
        out_shape=jax.ShapeDtypeStruct((h * s, s), jnp.float32),
    )(a_pad, pt, qh2)
    return out.reshape(b, h, s, s)


# speedup vs baseline: 2.0869x; 2.0869x over previous
import functools

import jax
import jax.numpy as jnp
from jax.experimental import pallas as pl


def _prep(qk):
    qk = qk.astype(jnp.float16)
    b, h, s, d = qk.shape
    qk_norm = jnp.sqrt(jnp.sum(qk * qk, axis=-1, keepdims=True))
    m = jnp.max(qk_norm)
    qk = qk * jnp.float16(0.75) / m
    tmp_zero = jnp.zeros(qk_norm.shape, dtype=jnp.float16)
    p = qk
    q = qk
    for i in range(2):
        tmp = jnp.float16(0.5) - jnp.power(qk_norm, (i + 1) * 2)
        p = jnp.concatenate((p, tmp), axis=-1)
        q = jnp.concatenate((q, tmp_zero), axis=-1)
    p_norm = jnp.sqrt(jnp.sum(p * p, axis=-1, keepdims=True))
    q_norm = jnp.sqrt(jnp.sum(q * q, axis=-1, keepdims=True))
    mm = jnp.max(p_norm)
    p = p / p_norm * mm
    q = q / q_norm * mm
    a = jax.random.normal(jax.random.key(42), (b, h, s, d + 2),
                          dtype=jnp.float32).astype(jnp.float16)
    qh = jnp.sum(q * a, axis=-1)
    return p, qh, a


_RB = 512
_K = 32


def _body(a_ref, pt_ref, qh_ref, o_ref):
    a_blk = a_ref[...]
    pt = pt_ref[...]
    qh = qh_ref[0:1, :]

    acc = jax.lax.dot_general(
        a_blk, pt, (((1,), (0,)), ((), ())),
        preferred_element_type=jnp.float32)
    s = acc * qh
    s = jnp.where(jnp.isnan(s), jnp.float32(0.0), s)
    s = s + jnp.float32(0.0)

    bits = jax.lax.bitcast_convert_type(s, jnp.int32)
    mono = jnp.where(bits < 0, ~bits, bits | jnp.int32(-2147483648))
    mono = jax.lax.bitcast_convert_type(mono, jnp.uint32)
    mono_s = jax.lax.bitcast_convert_type(mono, jnp.int32) ^ jnp.int32(-2147483648)

    rb, n = s.shape

    def count_ge(t):
        return jnp.sum((mono >= t).astype(jnp.int32), axis=1, keepdims=True)

    m = mono_s[:, 0:128]
    for k in range(1, 16):
        m = jnp.maximum(m, mono_s[:, k * 128:(k + 1) * 128])
    mm = jnp.maximum(jnp.maximum(m[:, 0:32], m[:, 32:64]),
                     jnp.maximum(m[:, 64:96], m[:, 96:128]))
    flip = jnp.int32(-2147483648)
    lo0 = jax.lax.bitcast_convert_type(
        jnp.min(mm, axis=1, keepdims=True) ^ flip, jnp.uint32)
    hi0 = jax.lax.bitcast_convert_type(
        jnp.max(m, axis=1, keepdims=True) ^ flip, jnp.uint32)

    def vcond(carry):
        lo, hi = carry
        return jnp.any(lo != hi)

    def vstep(carry):
        lo, hi = carry
        mid = lo + ((hi - lo + jnp.uint32(1)) >> 1)
        ge = count_ge(mid) >= _K
        return jnp.where(ge, mid, lo), jnp.where(ge, hi, mid - jnp.uint32(1))

    v32, _ = jax.lax.while_loop(vcond, vstep, (lo0, hi0))

    col = jax.lax.broadcasted_iota(jnp.int32, s.shape, 1)
    tie_rows = jnp.any(count_ge(v32) > _K)

    def tie_search():
        need = _K - jnp.sum((mono > v32).astype(jnp.int32),
                            axis=1, keepdims=True)
        tie = mono == v32

        def cstep(_, carry):
            lo, hi = carry
            mid = (lo + hi) >> 1
            cnt = jnp.sum((tie & (col <= mid)).astype(jnp.int32),
                          axis=1, keepdims=True)
            ge = cnt >= need
            return jnp.where(ge, lo, mid + 1), jnp.where(ge, mid, hi)

        _, c32 = jax.lax.fori_loop(0, 11, cstep,
                                   (jnp.zeros((rb, 1), jnp.int32),
                                    jnp.full((rb, 1), n - 1, jnp.int32)))
        return c32

    c32 = jax.lax.cond(tie_rows, tie_search,
                       lambda: jnp.full((rb, 1), n - 1, jnp.int32))
    sel = (mono > v32) | ((mono == v32) & (col <= c32))
    o_ref[...] = jnp.where(sel, jnp.float32(0.0), jnp.float32(-10000.0))


def kernel(qk, bucket_size):
    del bucket_size
    b, h, s, d = qk.shape
    p, qh, a = _prep(qk)
    pad = 128 - (d + 2)
    pt = jnp.transpose(
        jnp.pad(p[0], ((0, 0), (0, 0), (0, pad))), (0, 2, 1))
    pt = pt.reshape(h * 128, s).astype(jnp.float32)
    a_pad = jnp.pad(a[0], ((0, 0), (0, 0), (0, pad)))
    a_pad = a_pad.reshape(h * s, 128).astype(jnp.float32)
    qh2 = jnp.broadcast_to(qh[0][:, None, :],
                           (h, 8, s)).reshape(h * 8, s).astype(jnp.float32)

    nr = s // _RB
    out = pl.pallas_call(
        _body,
        grid=(h, nr),
        in_specs=[
            pl.BlockSpec((_RB, 128), lambda i, r: (i * nr + r, 0)),
            pl.BlockSpec((128, s), lambda i, r: (i, 0)),
            pl.BlockSpec((8, s), lambda i, r: (i, 0)),
        ],
        out_specs=pl.BlockSpec((_RB, s), lambda i, r: (i * nr + r, 0)),
        out_shape=jax.ShapeDtypeStruct((h * s, s), jnp.float32),
    )(a_pad, pt, qh2)
    return out.reshape(b, h, s, s)


# speedup vs baseline: 2.1030x; 1.0077x over previous
import functools

import jax
import jax.numpy as jnp
from jax.experimental import pallas as pl


def _prep(qk):
    qk = qk.astype(jnp.float16)
    b, h, s, d = qk.shape
    qk_norm = jnp.sqrt(jnp.sum(qk * qk, axis=-1, keepdims=True))
    m = jnp.max(qk_norm)
    qk = qk * jnp.float16(0.75) / m
    tmp_zero = jnp.zeros(qk_norm.shape, dtype=jnp.float16)
    p = qk
    q = qk
    for i in range(2):
        tmp = jnp.float16(0.5) - jnp.power(qk_norm, (i + 1) * 2)
        p = jnp.concatenate((p, tmp), axis=-1)
        q = jnp.concatenate((q, tmp_zero), axis=-1)
    p_norm = jnp.sqrt(jnp.sum(p * p, axis=-1, keepdims=True))
    q_norm = jnp.sqrt(jnp.sum(q * q, axis=-1, keepdims=True))
    mm = jnp.max(p_norm)
    p = p / p_norm * mm
    q = q / q_norm * mm
    a = jax.random.normal(jax.random.key(42), (b, h, s, d + 2),
                          dtype=jnp.float32).astype(jnp.float16)
    qh = jnp.sum(q * a, axis=-1)
    return p, qh, a


_RB = 1024
_K = 32


def _body(a_ref, pt_ref, qh_ref, o_ref):
    a_blk = a_ref[...]
    pt = pt_ref[...]
    qh = qh_ref[0:1, :]

    acc = jax.lax.dot_general(
        a_blk, pt, (((1,), (0,)), ((), ())),
        preferred_element_type=jnp.float32)
    s = acc * qh
    s = jnp.where(jnp.isnan(s), jnp.float32(0.0), s)
    s = s + jnp.float32(0.0)

    bits = jax.lax.bitcast_convert_type(s, jnp.int32)
    mono = jnp.where(bits < 0, ~bits, bits | jnp.int32(-2147483648))
    mono = jax.lax.bitcast_convert_type(mono, jnp.uint32)
    mono_s = jax.lax.bitcast_convert_type(mono, jnp.int32) ^ jnp.int32(-2147483648)

    rb, n = s.shape

    def count_ge(t):
        return jnp.sum((mono >= t).astype(jnp.int32), axis=1, keepdims=True)

    m = mono_s[:, 0:128]
    for k in range(1, 16):
        m = jnp.maximum(m, mono_s[:, k * 128:(k + 1) * 128])
    mm = jnp.maximum(jnp.maximum(m[:, 0:32], m[:, 32:64]),
                     jnp.maximum(m[:, 64:96], m[:, 96:128]))
    flip = jnp.int32(-2147483648)
    lo0 = jax.lax.bitcast_convert_type(
        jnp.min(mm, axis=1, keepdims=True) ^ flip, jnp.uint32)
    hi0 = jax.lax.bitcast_convert_type(
        jnp.max(m, axis=1, keepdims=True) ^ flip, jnp.uint32)

    def vcond(carry):
        lo, hi = carry
        return jnp.any(lo != hi)

    def vstep(carry):
        lo, hi = carry
        mid = lo + ((hi - lo + jnp.uint32(1)) >> 1)
        ge = count_ge(mid) >= _K
        return jnp.where(ge, mid, lo), jnp.where(ge, hi, mid - jnp.uint32(1))

    v32, _ = jax.lax.while_loop(vcond, vstep, (lo0, hi0))

    col = jax.lax.broadcasted_iota(jnp.int32, s.shape, 1)
    tie_rows = jnp.any(count_ge(v32) > _K)

    def tie_search():
        need = _K - jnp.sum((mono > v32).astype(jnp.int32),
                            axis=1, keepdims=True)
        tie = mono == v32

        def cstep(_, carry):
            lo, hi = carry
            mid = (lo + hi) >> 1
            cnt = jnp.sum((tie & (col <= mid)).astype(jnp.int32),
                          axis=1, keepdims=True)
            ge = cnt >= need
            return jnp.where(ge, lo, mid + 1), jnp.where(ge, mid, hi)

        _, c32 = jax.lax.fori_loop(0, 11, cstep,
                                   (jnp.zeros((rb, 1), jnp.int32),
                                    jnp.full((rb, 1), n - 1, jnp.int32)))
        return c32

    c32 = jax.lax.cond(tie_rows, tie_search,
                       lambda: jnp.full((rb, 1), n - 1, jnp.int32))
    sel = (mono > v32) | ((mono == v32) & (col <= c32))
    o_ref[...] = jnp.where(sel, jnp.float32(0.0), jnp.float32(-10000.0))


def kernel(qk, bucket_size):
    del bucket_size
    b, h, s, d = qk.shape
    p, qh, a = _prep(qk)
    pad = 128 - (d + 2)
    pt = jnp.transpose(
        jnp.pad(p[0], ((0, 0), (0, 0), (0, pad))), (0, 2, 1))
    pt = pt.reshape(h * 128, s).astype(jnp.float32)
    a_pad = jnp.pad(a[0], ((0, 0), (0, 0), (0, pad)))
    a_pad = a_pad.reshape(h * s, 128).astype(jnp.float32)
    qh2 = jnp.broadcast_to(qh[0][:, None, :],
                           (h, 8, s)).reshape(h * 8, s).astype(jnp.float32)

    nr = s // _RB
    out = pl.pallas_call(
        _body,
        grid=(h, nr),
        in_specs=[
            pl.BlockSpec((_RB, 128), lambda i, r: (i * nr + r, 0)),
            pl.BlockSpec((128, s), lambda i, r: (i, 0)),
            pl.BlockSpec((8, s), lambda i, r: (i, 0)),
        ],
        out_specs=pl.BlockSpec((_RB, s), lambda i, r: (i * nr + r, 0)),
        out_shape=jax.ShapeDtypeStruct((h * s, s), jnp.float32),
    )(a_pad, pt, qh2)
    return out.reshape(b, h, s, s)
